# baseline (speedup 1.0000x reference)
import functools

import jax
import jax.numpy as jnp
from jax import lax
from jax.experimental import pallas as pl
from jax.experimental.pallas import tpu as pltpu, tpu_sc as plsc

N = 10000
NP = 10240
G = 32
T = 16
F = 8
E = 320000
NW = 32
EPW = E // NW
CH = 128
NCHUNK = 80
NCH_T = 82
EPW_PAD = NCH_T * CH
PADE = EPW_PAD - EPW
ROWS_PER_TILE = NP // 16
HORIZON = 14
GRU_D = 32
POP = 1e10
BLK = 1000
GRID = N // BLK

_F32 = jnp.float32
_HIGH = jax.lax.Precision.HIGHEST


def _sigmoid(x):
    return 1.0 / (1.0 + jnp.exp(-x))


def _elu(x):
    return jnp.where(x > 0, x, jnp.exp(x) - 1.0)


def _proj_body(x_ref, w_ref, b_ref, aw_ref, ab_ref, z_ref, ws_ref, wd_ref):
    z = b_ref[...]
    for t in range(T):
        z = z + jnp.dot(x_ref[0, t], w_ref[t], preferred_element_type=_F32,
                        precision=_HIGH)
    wsd = jnp.dot(z, aw_ref[...], preferred_element_type=_F32,
                  precision=_HIGH) + ab_ref[...]
    z_ref[...] = z
    ws_ref[...] = wsd[:, 0:1]
    wd_ref[...] = wsd[:, 1:2]


def _proj_call(x, w, b, aw, ab):
    return pl.pallas_call(
        _proj_body,
        grid=(GRID,),
        in_specs=[
            pl.BlockSpec((1, T, BLK, F), lambda i: (0, 0, i, 0)),
            pl.BlockSpec((T, F, G), lambda i: (0, 0, 0)),
            pl.BlockSpec((1, G), lambda i: (0, 0)),
            pl.BlockSpec((G, 2), lambda i: (0, 0)),
            pl.BlockSpec((1, 2), lambda i: (0, 0)),
        ],
        out_specs=[
            pl.BlockSpec((BLK, G), lambda i: (i, 0)),
            pl.BlockSpec((BLK, 1), lambda i: (i, 0)),
            pl.BlockSpec((BLK, 1), lambda i: (i, 0)),
        ],
        out_shape=[
            jax.ShapeDtypeStruct((N, G), _F32),
            jax.ShapeDtypeStruct((N, 1), _F32),
            jax.ShapeDtypeStruct((N, 1), _F32),
        ],
    )(x, w, b, aw, ab)


def _layer2_body(p_ref, w_ref, b_ref, aw_ref, ab_ref, z_ref, ws_ref, wd_ref):
    h = _elu(p_ref[0] + p_ref[1])
    z = jnp.dot(h, w_ref[...], preferred_element_type=_F32,
                precision=_HIGH) + b_ref[...]
    wsd = jnp.dot(z, aw_ref[...], preferred_element_type=_F32,
                  precision=_HIGH) + ab_ref[...]
    z_ref[...] = z
    ws_ref[...] = wsd[:, 0:1]
    wd_ref[...] = wsd[:, 1:2]


def _layer2_call(p, w, b, aw, ab):
    return pl.pallas_call(
        _layer2_body,
        grid=(GRID,),
        in_specs=[
            pl.BlockSpec((2, BLK, G), lambda i: (0, i, 0)),
            pl.BlockSpec((G, G), lambda i: (0, 0)),
            pl.BlockSpec((1, G), lambda i: (0, 0)),
            pl.BlockSpec((G, 2), lambda i: (0, 0)),
            pl.BlockSpec((1, 2), lambda i: (0, 0)),
        ],
        out_specs=[
            pl.BlockSpec((BLK, G), lambda i: (i, 0)),
            pl.BlockSpec((BLK, 1), lambda i: (i, 0)),
            pl.BlockSpec((BLK, 1), lambda i: (i, 0)),
        ],
        out_shape=[
            jax.ShapeDtypeStruct((N, G), _F32),
            jax.ShapeDtypeStruct((N, 1), _F32),
            jax.ShapeDtypeStruct((N, 1), _F32),
        ],
    )(p, w, b, aw, ab)


_sc_mesh = plsc.VectorSubcoreMesh(core_axis_name="c", subcore_axis_name="s")


@functools.partial(
    pl.kernel,
    mesh=_sc_mesh,
    out_type=jax.ShapeDtypeStruct((2, NP, G), _F32),
    compiler_params=pltpu.CompilerParams(needs_layout_passes=False,
                                         use_tc_tiling_on_sc=False),
    scratch_types=[
        pltpu.VMEM((NP,), _F32),
        pltpu.VMEM((NP,), _F32),
        pltpu.VMEM((2, CH), jnp.int32),
        pltpu.VMEM((2, CH), jnp.int32),
        pltpu.VMEM((CH, G), _F32),
        pltpu.VMEM((CH, G), _F32),
        pltpu.VMEM((CH, G), _F32),
        pltpu.VMEM_SHARED((NP, G), _F32),
        pltpu.SemaphoreType.DMA,
        pltpu.SemaphoreType.DMA,
        pltpu.SemaphoreType.DMA,
        pltpu.SemaphoreType.DMA,
    ],
)
def _edge_kernel(z_hbm, ws_hbm, wd_hbm, edges_hbm, out_hbm,
                 ws_t, wd_t, eb0, eb1, zr0, zr1, scaled, acc,
                 si0, si1, sg0, sg1):
    c = lax.axis_index("c")
    s = lax.axis_index("s")
    wid = s * 2 + c
    cbase = wid * NCH_T

    pltpu.sync_copy(ws_hbm, ws_t.at[pl.ds(0, N)])
    pltpu.sync_copy(wd_hbm, wd_t.at[pl.ds(0, N)])
    zv = jnp.zeros((16,), _F32)
    for ei in range(CH):
        scaled[ei, pl.ds(0, 16)] = zv
        scaled[ei, pl.ds(16, 16)] = zv
    for r in range(ROWS_PER_TILE // CH):
        pltpu.sync_copy(scaled,
                        acc.at[pl.ds(s * ROWS_PER_TILE + r * CH, CH)])
    plsc.subcore_barrier()

    ebs = (eb0, eb1)
    zrs = (zr0, zr1)
    sis = (si0, si1)
    sgs = (sg0, sg1)

    def start_idx(k, b):
        pltpu.async_copy(edges_hbm.at[cbase + k], ebs[b], sis[b])

    def wait_idx(b):
        pltpu.make_async_copy(edges_hbm.at[cbase], ebs[b], sis[b]).wait()

    def start_gather(b):
        pltpu.async_copy(z_hbm.at[ebs[b].at[0]], zrs[b], sgs[b])

    def wait_gather(b):
        pltpu.make_async_copy(z_hbm.at[ebs[b].at[0]], zrs[b], sgs[b]).wait()

    def compute_and_scatter(b):
        eb = ebs[b]
        zr = zrs[b]
        for j in range(CH // 16):
            sv = eb[0, pl.ds(j * 16, 16)]
            dv = eb[1, pl.ds(j * 16, 16)]
            e = plsc.load_gather(ws_t, [sv]) + plsc.load_gather(wd_t, [dv])
            e = jnp.where(e >= 0, e, 0.01 * e)
            a = 1.0 / (1.0 + jnp.exp(-e))
            for m in range(16):
                asc = a[m]
                ei = j * 16 + m
                scaled[ei, pl.ds(0, 16)] = zr[ei, pl.ds(0, 16)] * asc
                scaled[ei, pl.ds(16, 16)] = zr[ei, pl.ds(16, 16)] * asc
        pltpu.sync_copy(scaled, acc.at[eb.at[1]], add=True)

    def chunk_body(k, b):
        wait_gather(b)
        wait_idx(1 - b)
        start_gather(1 - b)
        compute_and_scatter(b)
        start_idx(k + 2, b)

    start_idx(0, 0)
    start_idx(1, 1)
    wait_idx(0)
    start_gather(0)

    def pair(g, carry):
        chunk_body(2 * g, 0)
        chunk_body(2 * g + 1, 1)
        return carry

    lax.fori_loop(0, NCHUNK // 2, pair, 0)
    wait_gather(0)
    wait_idx(1)

    plsc.subcore_barrier()
    pltpu.sync_copy(acc.at[pl.ds(s * ROWS_PER_TILE, ROWS_PER_TILE)],
                    out_hbm.at[c, pl.ds(s * ROWS_PER_TILE, ROWS_PER_TILE)])


def _head_body(p_ref, x_ref, wih_ref, bi_ref, bh_ref, wh_ref, bhd_ref, st_ref,
               pred_ref, phy_ref):
    h2 = _elu(p_ref[0] + p_ref[1])
    gx = jnp.dot(h2, wih_ref[...], preferred_element_type=_F32,
                 precision=_HIGH) + bi_ref[...]
    bh = bh_ref[...]
    r = _sigmoid(gx[:, :GRU_D] + bh[:, :GRU_D])
    zg = _sigmoid(gx[:, GRU_D:2 * GRU_D] + bh[:, GRU_D:2 * GRU_D])
    ng = jnp.tanh(gx[:, 2 * GRU_D:] + r * bh[:, 2 * GRU_D:])
    h_out = (1.0 - zg) * ng
    xl = x_ref[0, 0]
    hc = jnp.concatenate([h_out, xl[:, 1:2], xl[:, 2:3]], axis=1)
    o = jnp.dot(hc, wh_ref[...], preferred_element_type=_F32,
                precision=_HIGH) + bhd_ref[...]
    pred_ref[...] = o[:, :2 * HORIZON]
    alpha = _sigmoid(o[:, 2 * HORIZON:2 * HORIZON + 1])
    beta = _sigmoid(o[:, 2 * HORIZON + 1:2 * HORIZON + 2])
    last_i = st_ref[:, 0:1]
    last_r = st_ref[:, 1:2]
    phy_i = []
    phy_r = []
    for _ in range(HORIZON):
        last_s = POP - last_i - last_r
        d_i = alpha * last_i * (last_s / POP) - beta * last_i
        d_r = beta * last_i
        phy_i.append(d_i)
        phy_r.append(d_r)
        last_i = last_i + d_i
        last_r = last_r + d_r
    phy_ref[...] = jnp.concatenate(phy_i + phy_r, axis=1)


def _head_call(p, x, wih, bi, bh, wh, bhd, st):
    return pl.pallas_call(
        _head_body,
        grid=(GRID,),
        in_specs=[
            pl.BlockSpec((2, BLK, G), lambda i: (0, i, 0)),
            pl.BlockSpec((1, 1, BLK, F), lambda i: (0, T - 1, i, 0)),
            pl.BlockSpec((GRU_D, 3 * GRU_D), lambda i: (0, 0)),
            pl.BlockSpec((1, 3 * GRU_D), lambda i: (0, 0)),
            pl.BlockSpec((1, 3 * GRU_D), lambda i: (0, 0)),
            pl.BlockSpec((GRU_D + 2, 2 * HORIZON + 2), lambda i: (0, 0)),
            pl.BlockSpec((1, 2 * HORIZON + 2), lambda i: (0, 0)),
            pl.BlockSpec((BLK, 2), lambda i: (i, 0)),
        ],
        out_specs=[
            pl.BlockSpec((BLK, 2 * HORIZON), lambda i: (i, 0)),
            pl.BlockSpec((BLK, 2 * HORIZON), lambda i: (i, 0)),
        ],
        out_shape=[
            jax.ShapeDtypeStruct((N, 2 * HORIZON), _F32),
            jax.ShapeDtypeStruct((N, 2 * HORIZON), _F32),
        ],
    )(p, x, wih, bi, bh, wh, bhd, st)


def _pack_edges(adj):
    spread = jnp.arange(NW * PADE, dtype=jnp.int32) % (NP - N)
    pads = (spread.reshape(NW, PADE), (N + spread).reshape(NW, PADE))
    out = []
    for x, pad in zip((adj[0], adj[1]), pads):
        x = jnp.concatenate([x.reshape(NW, EPW), pad], axis=1)
        out.append(x.reshape(NW, NCH_T, 1, CH))
    return jnp.concatenate(out, axis=2).reshape(NW * NCH_T, 2, CH)


def kernel(X, adj, states, l1_fc_w, l1_fc_b, l1_att_w, l1_att_b, l2_fc_w,
           l2_fc_b, l2_att_w, l2_att_b, gru_w_ih, gru_w_hh, gru_b_ih,
           gru_b_hh, res_I_w, res_I_b, res_R_w, res_R_b, sir_w, sir_b):
    edges = _pack_edges(adj)
    w4 = l1_fc_w.reshape(T, F, G)
    aw1 = jnp.concatenate([l1_att_w[:G], l1_att_w[G:]], axis=1)
    ab1 = jnp.stack([l1_att_b[0], jnp.zeros((), _F32)]).reshape(1, 2)
    aw2 = jnp.concatenate([l2_att_w[:G], l2_att_w[G:]], axis=1)
    ab2 = jnp.stack([l2_att_b[0], jnp.zeros((), _F32)]).reshape(1, 2)

    z1, ws1, wd1 = _proj_call(X, w4, l1_fc_b.reshape(1, G), aw1, ab1)
    p1 = _edge_kernel(z1, ws1.reshape(N), wd1.reshape(N), edges)

    z2, ws2, wd2 = _layer2_call(p1, l2_fc_w, l2_fc_b.reshape(1, G), aw2, ab2)
    p2 = _edge_kernel(z2, ws2.reshape(N), wd2.reshape(N), edges)

    wih = gru_w_ih.T
    bi = gru_b_ih.reshape(1, 3 * GRU_D)
    bh = gru_b_hh.reshape(1, 3 * GRU_D)
    wh = jnp.concatenate([res_I_w, res_R_w, sir_w], axis=1)
    bhd = jnp.concatenate([res_I_b, res_R_b, sir_b]).reshape(1, -1)

    o_pred, o_phy = _head_call(p2, X, wih, bi, bh, wh, bhd, states)

    pred = jnp.stack([o_pred[:, :HORIZON], o_pred[:, HORIZON:]], axis=-1)
    phy = jnp.stack([o_phy[:, :HORIZON], o_phy[:, HORIZON:]], axis=-1)
    return pred, phy


# speedup vs baseline: 1.3064x; 1.3064x over previous
import functools

import jax
import jax.numpy as jnp
from jax import lax
from jax.experimental import pallas as pl
from jax.experimental.pallas import tpu as pltpu, tpu_sc as plsc

N = 10000
NP = 10240
G = 32
T = 16
F = 8
E = 320000
NW = 32
EPW = E // NW
CH = 128
NCHUNK = 80
NCH_T = 82
EPW_PAD = NCH_T * CH
PADE = EPW_PAD - EPW
ROWS_PER_TILE = NP // 16
HORIZON = 14
GRU_D = 32
POP = 1e10
BLK = 1000
GRID = N // BLK

_F32 = jnp.float32
_HIGH = jax.lax.Precision.HIGHEST


def _sigmoid(x):
    return 1.0 / (1.0 + jnp.exp(-x))


def _elu(x):
    return jnp.where(x > 0, x, jnp.exp(x) - 1.0)


def _proj_body(x_ref, w_ref, b_ref, aw_ref, ab_ref, z_ref, ws_ref, wd_ref):
    z = jnp.dot(x_ref[...], w_ref[...], preferred_element_type=_F32,
                precision=_HIGH) + b_ref[...]
    wsd = jnp.dot(z, aw_ref[...], preferred_element_type=_F32,
                  precision=_HIGH) + ab_ref[...]
    z_ref[...] = z
    ws_ref[...] = wsd[:, 0:1]
    wd_ref[...] = wsd[:, 1:2]


def _proj_call(x, w, b, aw, ab):
    return pl.pallas_call(
        _proj_body,
        grid=(GRID,),
        in_specs=[
            pl.BlockSpec((BLK, T * F), lambda i: (i, 0)),
            pl.BlockSpec((T * F, G), lambda i: (0, 0)),
            pl.BlockSpec((1, G), lambda i: (0, 0)),
            pl.BlockSpec((G, 2), lambda i: (0, 0)),
            pl.BlockSpec((1, 2), lambda i: (0, 0)),
        ],
        out_specs=[
            pl.BlockSpec((BLK, G), lambda i: (i, 0)),
            pl.BlockSpec((BLK, 1), lambda i: (i, 0)),
            pl.BlockSpec((BLK, 1), lambda i: (i, 0)),
        ],
        out_shape=[
            jax.ShapeDtypeStruct((N, G), _F32),
            jax.ShapeDtypeStruct((N, 1), _F32),
            jax.ShapeDtypeStruct((N, 1), _F32),
        ],
    )(x, w, b, aw, ab)


def _layer2_body(p_ref, w_ref, b_ref, aw_ref, ab_ref, z_ref, ws_ref, wd_ref):
    h = _elu(p_ref[0] + p_ref[1])
    z = jnp.dot(h, w_ref[...], preferred_element_type=_F32,
                precision=_HIGH) + b_ref[...]
    wsd = jnp.dot(z, aw_ref[...], preferred_element_type=_F32,
                  precision=_HIGH) + ab_ref[...]
    z_ref[...] = z
    ws_ref[...] = wsd[:, 0:1]
    wd_ref[...] = wsd[:, 1:2]


def _layer2_call(p, w, b, aw, ab):
    return pl.pallas_call(
        _layer2_body,
        grid=(GRID,),
        in_specs=[
            pl.BlockSpec((2, BLK, G), lambda i: (0, i, 0)),
            pl.BlockSpec((G, G), lambda i: (0, 0)),
            pl.BlockSpec((1, G), lambda i: (0, 0)),
            pl.BlockSpec((G, 2), lambda i: (0, 0)),
            pl.BlockSpec((1, 2), lambda i: (0, 0)),
        ],
        out_specs=[
            pl.BlockSpec((BLK, G), lambda i: (i, 0)),
            pl.BlockSpec((BLK, 1), lambda i: (i, 0)),
            pl.BlockSpec((BLK, 1), lambda i: (i, 0)),
        ],
        out_shape=[
            jax.ShapeDtypeStruct((N, G), _F32),
            jax.ShapeDtypeStruct((N, 1), _F32),
            jax.ShapeDtypeStruct((N, 1), _F32),
        ],
    )(p, w, b, aw, ab)


_sc_mesh = plsc.VectorSubcoreMesh(core_axis_name="c", subcore_axis_name="s")


@functools.partial(
    pl.kernel,
    mesh=_sc_mesh,
    out_type=jax.ShapeDtypeStruct((2, NP, G), _F32),
    compiler_params=pltpu.CompilerParams(needs_layout_passes=False,
                                         use_tc_tiling_on_sc=False),
    scratch_types=[
        pltpu.VMEM((NP,), _F32),
        pltpu.VMEM((NP,), _F32),
        pltpu.VMEM((2, CH), jnp.int32),
        pltpu.VMEM((2, CH), jnp.int32),
        pltpu.VMEM((CH, G), _F32),
        pltpu.VMEM((CH, G), _F32),
        pltpu.VMEM((CH, G), _F32),
        pltpu.VMEM_SHARED((NP, G), _F32),
        pltpu.SemaphoreType.DMA,
        pltpu.SemaphoreType.DMA,
        pltpu.SemaphoreType.DMA,
        pltpu.SemaphoreType.DMA,
    ],
)
def _edge_kernel(z_hbm, ws_hbm, wd_hbm, edges_hbm, out_hbm,
                 ws_t, wd_t, eb0, eb1, zr0, zr1, scaled, acc,
                 si0, si1, sg0, sg1):
    c = lax.axis_index("c")
    s = lax.axis_index("s")
    wid = s * 2 + c
    cbase = wid * NCH_T

    pltpu.sync_copy(ws_hbm, ws_t.at[pl.ds(0, N)])
    pltpu.sync_copy(wd_hbm, wd_t.at[pl.ds(0, N)])
    zv = jnp.zeros((16,), _F32)
    for ei in range(CH):
        scaled[ei, pl.ds(0, 16)] = zv
        scaled[ei, pl.ds(16, 16)] = zv
    for r in range(ROWS_PER_TILE // CH):
        pltpu.sync_copy(scaled,
                        acc.at[pl.ds(s * ROWS_PER_TILE + r * CH, CH)])
    plsc.subcore_barrier()

    ebs = (eb0, eb1)
    zrs = (zr0, zr1)
    sis = (si0, si1)
    sgs = (sg0, sg1)

    def start_idx(k, b):
        pltpu.async_copy(edges_hbm.at[cbase + k], ebs[b], sis[b])

    def wait_idx(b):
        pltpu.make_async_copy(edges_hbm.at[cbase], ebs[b], sis[b]).wait()

    def start_gather(b):
        pltpu.async_copy(z_hbm.at[ebs[b].at[0]], zrs[b], sgs[b])

    def wait_gather(b):
        pltpu.make_async_copy(z_hbm.at[ebs[b].at[0]], zrs[b], sgs[b]).wait()

    def compute_and_scatter(b):
        eb = ebs[b]
        zr = zrs[b]
        for j in range(CH // 16):
            sv = eb[0, pl.ds(j * 16, 16)]
            dv = eb[1, pl.ds(j * 16, 16)]
            e = plsc.load_gather(ws_t, [sv]) + plsc.load_gather(wd_t, [dv])
            e = jnp.where(e >= 0, e, 0.01 * e)
            a = 1.0 / (1.0 + jnp.exp(-e))
            for m in range(16):
                asc = a[m]
                ei = j * 16 + m
                scaled[ei, pl.ds(0, 16)] = zr[ei, pl.ds(0, 16)] * asc
                scaled[ei, pl.ds(16, 16)] = zr[ei, pl.ds(16, 16)] * asc
        pltpu.sync_copy(scaled, acc.at[eb.at[1]], add=True)

    def chunk_body(k, b):
        wait_gather(b)
        wait_idx(1 - b)
        start_gather(1 - b)
        compute_and_scatter(b)
        start_idx(k + 2, b)

    start_idx(0, 0)
    start_idx(1, 1)
    wait_idx(0)
    start_gather(0)

    def pair(g, carry):
        chunk_body(2 * g, 0)
        chunk_body(2 * g + 1, 1)
        return carry

    lax.fori_loop(0, NCHUNK // 2, pair, 0)
    wait_gather(0)
    wait_idx(1)

    plsc.subcore_barrier()
    pltpu.sync_copy(acc.at[pl.ds(s * ROWS_PER_TILE, ROWS_PER_TILE)],
                    out_hbm.at[c, pl.ds(s * ROWS_PER_TILE, ROWS_PER_TILE)])


def _head_body(p_ref, ld_ref, wih_ref, bi_ref, bh_ref, wh_ref, bhd_ref,
               st_ref, pred_ref, phy_ref):
    h2 = _elu(p_ref[0] + p_ref[1])
    gx = jnp.dot(h2, wih_ref[...], preferred_element_type=_F32,
                 precision=_HIGH) + bi_ref[...]
    bh = bh_ref[...]
    r = _sigmoid(gx[:, :GRU_D] + bh[:, :GRU_D])
    zg = _sigmoid(gx[:, GRU_D:2 * GRU_D] + bh[:, GRU_D:2 * GRU_D])
    ng = jnp.tanh(gx[:, 2 * GRU_D:] + r * bh[:, 2 * GRU_D:])
    h_out = (1.0 - zg) * ng
    hc = jnp.concatenate([h_out, ld_ref[:, 0:1], ld_ref[:, 1:2]], axis=1)
    o = jnp.dot(hc, wh_ref[...], preferred_element_type=_F32,
                precision=_HIGH) + bhd_ref[...]
    pred_ref[...] = o[:, :2 * HORIZON]
    alpha = _sigmoid(o[:, 2 * HORIZON:2 * HORIZON + 1])
    beta = _sigmoid(o[:, 2 * HORIZON + 1:2 * HORIZON + 2])
    last_i = st_ref[:, 0:1]
    last_r = st_ref[:, 1:2]
    phy = []
    for _ in range(HORIZON):
        last_s = POP - last_i - last_r
        d_i = alpha * last_i * (last_s / POP) - beta * last_i
        d_r = beta * last_i
        phy.append(d_i)
        phy.append(d_r)
        last_i = last_i + d_i
        last_r = last_r + d_r
    phy_ref[...] = jnp.concatenate(phy, axis=1)


def _head_call(p, ld, wih, bi, bh, wh, bhd, st):
    return pl.pallas_call(
        _head_body,
        grid=(GRID,),
        in_specs=[
            pl.BlockSpec((2, BLK, G), lambda i: (0, i, 0)),
            pl.BlockSpec((BLK, 2), lambda i: (i, 0)),
            pl.BlockSpec((GRU_D, 3 * GRU_D), lambda i: (0, 0)),
            pl.BlockSpec((1, 3 * GRU_D), lambda i: (0, 0)),
            pl.BlockSpec((1, 3 * GRU_D), lambda i: (0, 0)),
            pl.BlockSpec((GRU_D + 2, 2 * HORIZON + 2), lambda i: (0, 0)),
            pl.BlockSpec((1, 2 * HORIZON + 2), lambda i: (0, 0)),
            pl.BlockSpec((BLK, 2), lambda i: (i, 0)),
        ],
        out_specs=[
            pl.BlockSpec((BLK, 2 * HORIZON), lambda i: (i, 0)),
            pl.BlockSpec((BLK, 2 * HORIZON), lambda i: (i, 0)),
        ],
        out_shape=[
            jax.ShapeDtypeStruct((N, 2 * HORIZON), _F32),
            jax.ShapeDtypeStruct((N, 2 * HORIZON), _F32),
        ],
    )(p, ld, wih, bi, bh, wh, bhd, st)


def _pack_edges(adj):
    spread = jnp.arange(NW * PADE, dtype=jnp.int32) % (NP - N)
    pads = (spread.reshape(NW, PADE), (N + spread).reshape(NW, PADE))
    out = []
    for x, pad in zip((adj[0], adj[1]), pads):
        x = jnp.concatenate([x.reshape(NW, EPW), pad], axis=1)
        out.append(x.reshape(NW, NCH_T, 1, CH))
    return jnp.concatenate(out, axis=2).reshape(NW * NCH_T, 2, CH)


def kernel(X, adj, states, l1_fc_w, l1_fc_b, l1_att_w, l1_att_b, l2_fc_w,
           l2_fc_b, l2_att_w, l2_att_b, gru_w_ih, gru_w_hh, gru_b_ih,
           gru_b_hh, res_I_w, res_I_b, res_R_w, res_R_b, sir_w, sir_b):
    edges = _pack_edges(adj)
    Xf = jnp.transpose(X, (0, 2, 1, 3)).reshape(N, T * F)
    ld = X[0, -1, :, 1:3]
    aw1 = jnp.concatenate([l1_att_w[:G], l1_att_w[G:]], axis=1)
    ab1 = jnp.stack([l1_att_b[0], jnp.zeros((), _F32)]).reshape(1, 2)
    aw2 = jnp.concatenate([l2_att_w[:G], l2_att_w[G:]], axis=1)
    ab2 = jnp.stack([l2_att_b[0], jnp.zeros((), _F32)]).reshape(1, 2)

    z1, ws1, wd1 = _proj_call(Xf, l1_fc_w, l1_fc_b.reshape(1, G), aw1, ab1)
    p1 = _edge_kernel(z1, ws1.reshape(N), wd1.reshape(N), edges)

    z2, ws2, wd2 = _layer2_call(p1, l2_fc_w, l2_fc_b.reshape(1, G), aw2, ab2)
    p2 = _edge_kernel(z2, ws2.reshape(N), wd2.reshape(N), edges)

    wih = gru_w_ih.T
    bi = gru_b_ih.reshape(1, 3 * GRU_D)
    bh = gru_b_hh.reshape(1, 3 * GRU_D)
    wir = jnp.stack([res_I_w, res_R_w], axis=-1).reshape(GRU_D + 2,
                                                         2 * HORIZON)
    bir = jnp.stack([res_I_b, res_R_b], axis=-1).reshape(2 * HORIZON)
    wh = jnp.concatenate([wir, sir_w], axis=1)
    bhd = jnp.concatenate([bir, sir_b]).reshape(1, -1)

    o_pred, o_phy = _head_call(p2, ld, wih, bi, bh, wh, bhd, states)

    pred = o_pred.reshape(N, HORIZON, 2)
    phy = o_phy.reshape(N, HORIZON, 2)
    return pred, phy
